# Initial kernel scaffold; baseline (speedup 1.0000x reference)
import jax
import jax.numpy as jnp
from jax.experimental import pallas as pl


def kernel(input_ids, weight):
    raise NotImplementedError("write your pallas kernel here")


# speedup vs baseline: 1.8755x; 1.8755x over previous
import jax
import jax.numpy as jnp
from jax import lax
from jax.experimental import pallas as pl
from jax.experimental.pallas import tpu as pltpu
from jax.experimental.pallas import tpu_sc as plsc

VOCAB = 1000000
EMB = 64
B_TOTAL = 16384 * 50

NC, NS = 2, 16
NW = NC * NS
B_PER_W = B_TOTAL // NW
CHUNK = 128
NCHUNK = B_PER_W // CHUNK
NBUF = 4


def _emb_kernel(ids_hbm, table_hbm, out_hbm, idx_v, rows_v, gsems):
    wid = lax.axis_index("c") * NS + lax.axis_index("s")
    base_w = wid * B_PER_W

    pltpu.sync_copy(ids_hbm.at[wid], idx_v)

    def start_gather(g, b):
        return pltpu.async_copy(table_hbm.at[idx_v.at[g]], rows_v.at[b],
                                gsems.at[b])

    def finish_chunk(g, b):
        pltpu.make_async_copy(table_hbm.at[idx_v.at[g]], rows_v.at[b],
                              gsems.at[b]).wait()
        pltpu.sync_copy(rows_v.at[b], out_hbm.at[pl.ds(base_w + g * CHUNK,
                                                       CHUNK)])

    for b in range(NBUF):
        start_gather(b, b)

    def outer(k):
        for b in range(NBUF):
            g = k * NBUF + b
            finish_chunk(g, b)
            start_gather(g + NBUF, b)

    pl.loop(0, (NCHUNK - NBUF) // NBUF)(outer)

    for b in range(NBUF):
        g = NCHUNK - NBUF + b
        finish_chunk(g, b)


def kernel(input_ids, weight):
    ids = input_ids.reshape(NW, NCHUNK, CHUNK).astype(jnp.int32)
    mesh = plsc.VectorSubcoreMesh(core_axis_name="c", subcore_axis_name="s")
    out = pl.kernel(
        _emb_kernel,
        mesh=mesh,
        compiler_params=pltpu.CompilerParams(use_tc_tiling_on_sc=False),
        out_type=jax.ShapeDtypeStruct((B_TOTAL, EMB), jnp.float32),
        scratch_types=[
            pltpu.VMEM((NCHUNK, CHUNK), jnp.int32),
            pltpu.VMEM((NBUF, CHUNK, EMB), jnp.float32),
            pltpu.SemaphoreType.DMA((NBUF,)),
        ],
    )(ids, weight)
    return out.reshape(input_ids.shape + (EMB,))
